# Initial kernel scaffold; baseline (speedup 1.0000x reference)
import jax
import jax.numpy as jnp
from jax.experimental import pallas as pl


def kernel(x_num, x_cat, weight, bias, cat_emb):
    raise NotImplementedError("write your pallas kernel here")


# speedup vs baseline: 2.4371x; 2.4371x over previous
import functools

import jax
import jax.numpy as jnp
from jax import lax
from jax.experimental import pallas as pl
from jax.experimental.pallas import tpu as pltpu
from jax.experimental.pallas import tpu_sc as plsc

B = 16384
D_NUM = 13
N_CAT = 26
CARD = 1000
D_TOKEN = 64
TOTAL_CAT = N_CAT * CARD
N_TOK = 1 + D_NUM + N_CAT

NUM_COLS = (1 + D_NUM) * D_TOKEN
ROW_COLS = N_TOK * D_TOKEN

SC_CORES = 2
SC_SUBCORES = 16
NW = SC_CORES * SC_SUBCORES
ROWS_PER_W = B // NW
CHUNK_ROWS = 16
N_CHUNKS = ROWS_PER_W // CHUNK_ROWS
CHUNK_IDX = CHUNK_ROWS * N_CAT


def _prep_kernel(emb_ref, bias_ref, xcat_ref, tab_ref, idx_ref):
    tab_ref[...] = emb_ref[...] + bias_ref[...]
    offs = lax.broadcasted_iota(jnp.int32, (B, N_CAT), 1) * CARD
    idx_ref[...] = xcat_ref[...] + offs


def _prep(cat_emb, bias_cat, x_cat):
    emb3 = cat_emb.reshape(N_CAT, CARD, D_TOKEN)
    bias3 = bias_cat.reshape(N_CAT, 1, D_TOKEN)
    tab3, idx = pl.pallas_call(
        _prep_kernel,
        out_shape=(
            jax.ShapeDtypeStruct((N_CAT, CARD, D_TOKEN), jnp.float32),
            jax.ShapeDtypeStruct((B, N_CAT), jnp.int32),
        ),
    )(emb3, bias3, x_cat)
    return tab3.reshape(TOTAL_CAT, D_TOKEN), idx.reshape(B * N_CAT)


def _sc_body(tab_ref, idx_ref, out_ref, idx_v, rows_v, sem):
    wid = lax.axis_index("s") * SC_CORES + lax.axis_index("c")
    base = wid * ROWS_PER_W

    def chunk(g, carry):
        r0 = base + g * CHUNK_ROWS
        pltpu.sync_copy(idx_ref.at[pl.ds(r0 * N_CAT, CHUNK_IDX)], idx_v)
        pltpu.async_copy(tab_ref.at[idx_v], rows_v, sem).wait()
        for r in range(CHUNK_ROWS):
            pltpu.sync_copy(
                rows_v.at[pl.ds(r * N_CAT, N_CAT)],
                out_ref.at[r0 + r, pl.ds(1 + D_NUM, N_CAT)],
            )
        return carry

    lax.fori_loop(0, N_CHUNKS, chunk, 0)


def _sc_gather(table, idx_flat):
    mesh = plsc.VectorSubcoreMesh(core_axis_name="c", subcore_axis_name="s")
    f = pl.kernel(
        _sc_body,
        out_type=jax.ShapeDtypeStruct((B, N_TOK, D_TOKEN), jnp.float32),
        mesh=mesh,
        scratch_types=[
            pltpu.VMEM((CHUNK_IDX,), jnp.int32),
            pltpu.VMEM((CHUNK_IDX, D_TOKEN), jnp.float32),
            pltpu.SemaphoreType.DMA,
        ],
        compiler_params=pltpu.CompilerParams(use_tc_tiling_on_sc=False),
    )
    return f(table, idx_flat)


def _num_kernel(x_ref, w_ref, b_ref, alias_ref, o_ref):
    del alias_ref
    o_ref[...] = (
        jnp.dot(
            x_ref[...],
            w_ref[...],
            preferred_element_type=jnp.float32,
            precision=lax.Precision.HIGHEST,
        )
        + b_ref[...]
    )


def _num_finish(x_num, ew, bnew, out0_2d):
    bs = 512
    return pl.pallas_call(
        _num_kernel,
        grid=(B // bs,),
        in_specs=[
            pl.BlockSpec((bs, D_NUM), lambda i: (i, 0)),
            pl.BlockSpec((D_NUM, NUM_COLS), lambda i: (0, 0)),
            pl.BlockSpec((1, NUM_COLS), lambda i: (0, 0)),
            pl.BlockSpec(memory_space=pl.ANY),
        ],
        out_specs=pl.BlockSpec((bs, NUM_COLS), lambda i: (i, 0)),
        out_shape=jax.ShapeDtypeStruct((B, ROW_COLS), jnp.float32),
        input_output_aliases={3: 0},
    )(x_num, ew, bnew, out0_2d)


def kernel(x_num, x_cat, weight, bias, cat_emb):
    w_num = weight[1:]
    ew = (jnp.eye(D_NUM, dtype=jnp.float32)[:, :, None] * w_num[:, None, :]).reshape(
        D_NUM, D_NUM * D_TOKEN
    )
    ew = jnp.concatenate([jnp.zeros((D_NUM, D_TOKEN), jnp.float32), ew], axis=1)
    bias_num = jnp.concatenate([weight[0:1], bias[:D_NUM]], axis=0).reshape(
        1, NUM_COLS
    )
    bias_cat = bias[D_NUM:]

    table, idx_flat = _prep(cat_emb, bias_cat, x_cat)
    out0 = _sc_gather(table, idx_flat)
    out2d = _num_finish(x_num, ew, bias_num, out0.reshape(B, ROW_COLS))
    return out2d.reshape(B, N_TOK, D_TOKEN)
